# Initial kernel scaffold; baseline (speedup 1.0000x reference)
import functools

import jax
import jax.numpy as jnp
from jax import lax
from jax.experimental import pallas as pl
from jax.experimental.pallas import tpu as pltpu
from jax.experimental.pallas import tpu_sc as plsc

N = 10000
E = 320000
D = 128
N_CLS = 40

NC = 2
NS = 16
NW = NC * NS
C = 128
NCH = -(-E // (NW * C))
EPAD = NW * NCH * C
NPAD = 10016
ZROWS = NPAD // NS
ROWS_OUT = N // NS

ROW_BLK = 1000
GRID = N // ROW_BLK


def _zero_vmem_rows(ref, nrows, width):
    def body(i, carry):
        for j in range(width // 16):
            ref[i, pl.ds(j * 16, 16)] = jnp.zeros((16,), jnp.float32)
        return carry
    lax.fori_loop(0, nrows, body, 0)


def _fill_ones_rows(ref, nrows, width):
    def body(i, carry):
        for j in range(width // 16):
            ref[i, pl.ds(j * 16, 16)] = jnp.ones((16,), jnp.float32)
        return carry
    lax.fori_loop(0, nrows, body, 0)


def _segsum_body(g_hbm, src_hbm, dst_hbm, out_hbm, src_v, dst_v, rows_v, acc, sem):
    cid = lax.axis_index("c")
    sid = lax.axis_index("s")
    wid = cid * NS + sid

    _zero_vmem_rows(rows_v, C, D)
    base = sid * ZROWS
    done = 0
    while done < ZROWS:
        cnt = min(C, ZROWS - done)
        pltpu.sync_copy(rows_v.at[pl.ds(0, cnt)], acc.at[pl.ds(base + done, cnt)])
        done += cnt

    pltpu.sync_copy(src_hbm.at[wid], src_v)
    pltpu.sync_copy(dst_hbm.at[wid], dst_v)

    plsc.subcore_barrier()

    def body(j, carry):
        pltpu.async_copy(g_hbm.at[src_v.at[j]], rows_v, sem).wait()
        pltpu.sync_copy(rows_v, acc.at[dst_v.at[j]], add=True)
        return carry
    lax.fori_loop(0, NCH, body, 0)

    plsc.subcore_barrier()

    pltpu.sync_copy(acc.at[pl.ds(sid * ROWS_OUT, ROWS_OUT)],
                    out_hbm.at[cid, pl.ds(sid * ROWS_OUT, ROWS_OUT)])


_segsum = functools.partial(
    pl.kernel,
    out_type=jax.ShapeDtypeStruct((NC, N, D), jnp.float32),
    mesh=plsc.VectorSubcoreMesh(core_axis_name="c", subcore_axis_name="s"),
    scratch_types=[
        pltpu.VMEM((NCH, C), jnp.int32),
        pltpu.VMEM((NCH, C), jnp.int32),
        pltpu.VMEM((C, D), jnp.float32),
        pltpu.VMEM_SHARED((NPAD, D), jnp.float32),
        pltpu.SemaphoreType.DMA,
    ],
)(_segsum_body)


def _deg_body(dst_hbm, out_hbm, dst_v, ones_v, zero_v, acc):
    cid = lax.axis_index("c")
    sid = lax.axis_index("s")
    wid = cid * NS + sid

    _zero_vmem_rows(zero_v, C, 16)
    _fill_ones_rows(ones_v, C, 16)
    base = sid * ZROWS
    done = 0
    while done < ZROWS:
        cnt = min(C, ZROWS - done)
        pltpu.sync_copy(zero_v.at[pl.ds(0, cnt)], acc.at[pl.ds(base + done, cnt)])
        done += cnt

    pltpu.sync_copy(dst_hbm.at[wid], dst_v)

    plsc.subcore_barrier()

    def body(j, carry):
        pltpu.sync_copy(ones_v, acc.at[dst_v.at[j]], add=True)
        return carry
    lax.fori_loop(0, NCH, body, 0)

    plsc.subcore_barrier()

    pltpu.sync_copy(acc.at[pl.ds(sid * ROWS_OUT, ROWS_OUT)],
                    out_hbm.at[cid, pl.ds(sid * ROWS_OUT, ROWS_OUT)])


_deg = functools.partial(
    pl.kernel,
    out_type=jax.ShapeDtypeStruct((NC, N, 16), jnp.float32),
    mesh=plsc.VectorSubcoreMesh(core_axis_name="c", subcore_axis_name="s"),
    scratch_types=[
        pltpu.VMEM((NCH, C), jnp.int32),
        pltpu.VMEM((C, 16), jnp.float32),
        pltpu.VMEM((C, 16), jnp.float32),
        pltpu.VMEM_SHARED((NPAD, 16), jnp.float32),
    ],
)(_deg_body)


def _proj0_body(x_ref, wn_ref, pdeg_ref, g_ref, inv_ref):
    g_ref[...] = jnp.dot(x_ref[...], wn_ref[...],
                         preferred_element_type=jnp.float32)
    d = pdeg_ref[0] + pdeg_ref[1]
    inv_ref[...] = 1.0 / jnp.maximum(d, 1.0)


def _proj0(x, wn0, pdeg):
    return pl.pallas_call(
        _proj0_body,
        grid=(GRID,),
        in_specs=[
            pl.BlockSpec((ROW_BLK, D), lambda i: (i, 0)),
            pl.BlockSpec((D, D), lambda i: (0, 0)),
            pl.BlockSpec((NC, ROW_BLK, 16), lambda i: (0, i, 0)),
        ],
        out_specs=[
            pl.BlockSpec((ROW_BLK, D), lambda i: (i, 0)),
            pl.BlockSpec((ROW_BLK, 16), lambda i: (i, 0)),
        ],
        out_shape=[
            jax.ShapeDtypeStruct((N, D), jnp.float32),
            jax.ShapeDtypeStruct((N, 16), jnp.float32),
        ],
    )(x, wn0, pdeg)


def _combine_body(h_ref, p_ref, inv_ref, ws_ref, b_ref, wn_ref, h_out, g_out):
    agg = (p_ref[0] + p_ref[1]) * inv_ref[:, 0:1]
    t = jnp.dot(h_ref[...], ws_ref[...], preferred_element_type=jnp.float32)
    t = jnp.maximum(t + agg + b_ref[...], 0.0)
    h_out[...] = t
    g_out[...] = jnp.dot(t, wn_ref[...], preferred_element_type=jnp.float32)


def _combine(h, p, inv16, ws, b, wn_next):
    return pl.pallas_call(
        _combine_body,
        grid=(GRID,),
        in_specs=[
            pl.BlockSpec((ROW_BLK, D), lambda i: (i, 0)),
            pl.BlockSpec((NC, ROW_BLK, D), lambda i: (0, i, 0)),
            pl.BlockSpec((ROW_BLK, 16), lambda i: (i, 0)),
            pl.BlockSpec((D, D), lambda i: (0, 0)),
            pl.BlockSpec((1, D), lambda i: (0, 0)),
            pl.BlockSpec((D, D), lambda i: (0, 0)),
        ],
        out_specs=[
            pl.BlockSpec((ROW_BLK, D), lambda i: (i, 0)),
            pl.BlockSpec((ROW_BLK, D), lambda i: (i, 0)),
        ],
        out_shape=[
            jax.ShapeDtypeStruct((N, D), jnp.float32),
            jax.ShapeDtypeStruct((N, D), jnp.float32),
        ],
    )(h, p, inv16, ws, b.reshape(1, D), wn_next)


def _combine_last_body(h_ref, p_ref, inv_ref, ws_ref, b_ref, psum_out):
    agg = (p_ref[0] + p_ref[1]) * inv_ref[:, 0:1]
    t = jnp.dot(h_ref[...], ws_ref[...], preferred_element_type=jnp.float32)
    t = jnp.maximum(t + agg + b_ref[...], 0.0)
    psum_out[...] = jnp.sum(t, axis=0, keepdims=True)


def _combine_last(h, p, inv16, ws, b):
    return pl.pallas_call(
        _combine_last_body,
        grid=(GRID,),
        in_specs=[
            pl.BlockSpec((ROW_BLK, D), lambda i: (i, 0)),
            pl.BlockSpec((NC, ROW_BLK, D), lambda i: (0, i, 0)),
            pl.BlockSpec((ROW_BLK, 16), lambda i: (i, 0)),
            pl.BlockSpec((D, D), lambda i: (0, 0)),
            pl.BlockSpec((1, D), lambda i: (0, 0)),
        ],
        out_specs=pl.BlockSpec((1, D), lambda i: (i, 0)),
        out_shape=jax.ShapeDtypeStruct((GRID, D), jnp.float32),
    )(h, p, inv16, ws, b.reshape(1, D))


def _head_body(ps_ref, w1_ref, b1_ref, w2_ref, b2_ref, out_ref):
    m = jnp.sum(ps_ref[...], axis=0, keepdims=True) * (1.0 / N)
    y = jnp.dot(m, w1_ref[...], preferred_element_type=jnp.float32) + b1_ref[...]
    y = jnp.where(y > 0, y, jnp.exp(y) - 1.0)
    z = jnp.dot(y, w2_ref[...], preferred_element_type=jnp.float32) + b2_ref[...]
    mx = jnp.max(z, axis=0, keepdims=True)
    e = z - mx
    out_ref[...] = e - jnp.log(jnp.sum(jnp.exp(e), axis=0, keepdims=True))


def _head(psums, w1, b1, w2, b2):
    return pl.pallas_call(
        _head_body,
        out_shape=jax.ShapeDtypeStruct((1, N_CLS), jnp.float32),
    )(psums, w1, b1.reshape(1, D), w2, b2.reshape(1, N_CLS))


def kernel(x, edge_index, Ws0, Wn0, b0, Ws1, Wn1, b1, Ws2, Wn2, b2,
           Ws3, Wn3, b3, W_fc1, b_fc1, W_fc2, b_fc2):
    src = edge_index[0].astype(jnp.int32)
    dst = edge_index[1].astype(jnp.int32)
    pad = EPAD - E
    src3 = jnp.concatenate([src, jnp.zeros((pad,), jnp.int32)]).reshape(NW, NCH, C)
    dst3 = jnp.concatenate([dst, jnp.full((pad,), N, jnp.int32)]).reshape(NW, NCH, C)

    pdeg = _deg(dst3)
    g, inv16 = _proj0(x, Wn0, pdeg)

    h = x
    layers = [(Ws0, b0, Wn1), (Ws1, b1, Wn2), (Ws2, b2, Wn3)]
    for (ws, b, wn_next) in layers:
        p = _segsum(g, src3, dst3)
        h, g = _combine(h, p, inv16, ws, b, wn_next)
    p = _segsum(g, src3, dst3)
    psums = _combine_last(h, p, inv16, Ws3, b3)

    return _head(psums, W_fc1, b_fc1, W_fc2, b_fc2)


# speedup vs baseline: 4.9515x; 4.9515x over previous
import functools

import jax
import jax.numpy as jnp
from jax import lax
from jax.experimental import pallas as pl
from jax.experimental.pallas import tpu as pltpu
from jax.experimental.pallas import tpu_sc as plsc

N = 10000
E = 320000
D = 128
N_CLS = 40

NC = 2
NS = 16
NW = NC * NS
C = 128
NCH = -(-E // (NW * C))
EPAD = NW * NCH * C
NPAD = 10112
ZROWS = NPAD // NS
ROWS_OUT = 624

ROW_BLK = 1000
GRID = N // ROW_BLK


def _zero_vmem_rows(ref, nrows, width):
    def body(i, carry):
        for j in range(width // 16):
            ref[i, pl.ds(j * 16, 16)] = jnp.zeros((16,), jnp.float32)
        return carry
    lax.fori_loop(0, nrows, body, 0)


def _fill_ones_rows(ref, nrows, width):
    def body(i, carry):
        for j in range(width // 16):
            ref[i, pl.ds(j * 16, 16)] = jnp.ones((16,), jnp.float32)
        return carry
    lax.fori_loop(0, nrows, body, 0)


def _segsum_body(g_hbm, src_hbm, dst_hbm, out_hbm, src_v, dst_v, rows_v, acc, sem):
    cid = lax.axis_index("c")
    sid = lax.axis_index("s")
    wid = cid * NS + sid

    _zero_vmem_rows(rows_v, C, D)
    base = sid * ZROWS
    done = 0
    while done < ZROWS:
        cnt = min(C, ZROWS - done)
        pltpu.sync_copy(rows_v.at[pl.ds(0, cnt)], acc.at[pl.ds(base + done, cnt)])
        done += cnt

    pltpu.sync_copy(src_hbm.at[wid], src_v)
    pltpu.sync_copy(dst_hbm.at[wid], dst_v)

    plsc.subcore_barrier()

    def body(j, carry):
        pltpu.async_copy(g_hbm.at[src_v.at[j]], rows_v, sem).wait()
        pltpu.sync_copy(rows_v, acc.at[dst_v.at[j]], add=True)
        return carry
    lax.fori_loop(0, NCH, body, 0)

    plsc.subcore_barrier()

    pltpu.sync_copy(acc.at[pl.ds(sid * ROWS_OUT, ROWS_OUT)],
                    out_hbm.at[cid, pl.ds(sid * ROWS_OUT, ROWS_OUT)])

    @pl.when(sid == NS - 1)
    def _():
        tail = NS * ROWS_OUT
        pltpu.sync_copy(acc.at[pl.ds(tail, N - tail)],
                        out_hbm.at[cid, pl.ds(tail, N - tail)])


_segsum = functools.partial(
    pl.kernel,
    out_type=jax.ShapeDtypeStruct((NC, N, D), jnp.float32),
    mesh=plsc.VectorSubcoreMesh(core_axis_name="c", subcore_axis_name="s"),
    scratch_types=[
        pltpu.VMEM((NCH, C), jnp.int32),
        pltpu.VMEM((NCH, C), jnp.int32),
        pltpu.VMEM((C, D), jnp.float32),
        pltpu.VMEM_SHARED((NPAD, D), jnp.float32),
        pltpu.SemaphoreType.DMA,
    ],
)(_segsum_body)


def _deg_body(dst_hbm, out_hbm, dst_v, ones_v, zero_v, acc):
    cid = lax.axis_index("c")
    sid = lax.axis_index("s")
    wid = cid * NS + sid

    _zero_vmem_rows(zero_v, C, 16)
    _fill_ones_rows(ones_v, C, 16)
    base = sid * ZROWS
    done = 0
    while done < ZROWS:
        cnt = min(C, ZROWS - done)
        pltpu.sync_copy(zero_v.at[pl.ds(0, cnt)], acc.at[pl.ds(base + done, cnt)])
        done += cnt

    pltpu.sync_copy(dst_hbm.at[wid], dst_v)

    plsc.subcore_barrier()

    def body(j, carry):
        pltpu.sync_copy(ones_v, acc.at[dst_v.at[j]], add=True)
        return carry
    lax.fori_loop(0, NCH, body, 0)

    plsc.subcore_barrier()

    pltpu.sync_copy(acc.at[pl.ds(sid * ROWS_OUT, ROWS_OUT)],
                    out_hbm.at[cid, pl.ds(sid * ROWS_OUT, ROWS_OUT)])

    @pl.when(sid == NS - 1)
    def _():
        tail = NS * ROWS_OUT
        pltpu.sync_copy(acc.at[pl.ds(tail, N - tail)],
                        out_hbm.at[cid, pl.ds(tail, N - tail)])


_deg = functools.partial(
    pl.kernel,
    out_type=jax.ShapeDtypeStruct((NC, N, 16), jnp.float32),
    mesh=plsc.VectorSubcoreMesh(core_axis_name="c", subcore_axis_name="s"),
    scratch_types=[
        pltpu.VMEM((NCH, C), jnp.int32),
        pltpu.VMEM((C, 16), jnp.float32),
        pltpu.VMEM((C, 16), jnp.float32),
        pltpu.VMEM_SHARED((NPAD, 16), jnp.float32),
    ],
)(_deg_body)


def _proj0_body(x_ref, wn_ref, pdeg_ref, g_ref, inv_ref):
    g_ref[...] = jnp.dot(x_ref[...], wn_ref[...],
                         preferred_element_type=jnp.float32)
    d = pdeg_ref[0] + pdeg_ref[1]
    inv_ref[...] = 1.0 / jnp.maximum(d, 1.0)


def _proj0(x, wn0, pdeg):
    return pl.pallas_call(
        _proj0_body,
        grid=(GRID,),
        in_specs=[
            pl.BlockSpec((ROW_BLK, D), lambda i: (i, 0)),
            pl.BlockSpec((D, D), lambda i: (0, 0)),
            pl.BlockSpec((NC, ROW_BLK, 16), lambda i: (0, i, 0)),
        ],
        out_specs=[
            pl.BlockSpec((ROW_BLK, D), lambda i: (i, 0)),
            pl.BlockSpec((ROW_BLK, 16), lambda i: (i, 0)),
        ],
        out_shape=[
            jax.ShapeDtypeStruct((N, D), jnp.float32),
            jax.ShapeDtypeStruct((N, 16), jnp.float32),
        ],
    )(x, wn0, pdeg)


def _combine_body(h_ref, p_ref, inv_ref, ws_ref, b_ref, wn_ref, h_out, g_out):
    agg = (p_ref[0] + p_ref[1]) * inv_ref[:, 0:1]
    t = jnp.dot(h_ref[...], ws_ref[...], preferred_element_type=jnp.float32)
    t = jnp.maximum(t + agg + b_ref[...], 0.0)
    h_out[...] = t
    g_out[...] = jnp.dot(t, wn_ref[...], preferred_element_type=jnp.float32)


def _combine(h, p, inv16, ws, b, wn_next):
    return pl.pallas_call(
        _combine_body,
        grid=(GRID,),
        in_specs=[
            pl.BlockSpec((ROW_BLK, D), lambda i: (i, 0)),
            pl.BlockSpec((NC, ROW_BLK, D), lambda i: (0, i, 0)),
            pl.BlockSpec((ROW_BLK, 16), lambda i: (i, 0)),
            pl.BlockSpec((D, D), lambda i: (0, 0)),
            pl.BlockSpec((1, D), lambda i: (0, 0)),
            pl.BlockSpec((D, D), lambda i: (0, 0)),
        ],
        out_specs=[
            pl.BlockSpec((ROW_BLK, D), lambda i: (i, 0)),
            pl.BlockSpec((ROW_BLK, D), lambda i: (i, 0)),
        ],
        out_shape=[
            jax.ShapeDtypeStruct((N, D), jnp.float32),
            jax.ShapeDtypeStruct((N, D), jnp.float32),
        ],
    )(h, p, inv16, ws, b.reshape(1, D), wn_next)


def _combine_last_body(h_ref, p_ref, inv_ref, ws_ref, b_ref, psum_out):
    agg = (p_ref[0] + p_ref[1]) * inv_ref[:, 0:1]
    t = jnp.dot(h_ref[...], ws_ref[...], preferred_element_type=jnp.float32)
    t = jnp.maximum(t + agg + b_ref[...], 0.0)
    psum_out[...] = jnp.sum(t.reshape(8, ROW_BLK // 8, D), axis=1)[None]


def _combine_last(h, p, inv16, ws, b):
    return pl.pallas_call(
        _combine_last_body,
        grid=(GRID,),
        in_specs=[
            pl.BlockSpec((ROW_BLK, D), lambda i: (i, 0)),
            pl.BlockSpec((NC, ROW_BLK, D), lambda i: (0, i, 0)),
            pl.BlockSpec((ROW_BLK, 16), lambda i: (i, 0)),
            pl.BlockSpec((D, D), lambda i: (0, 0)),
            pl.BlockSpec((1, D), lambda i: (0, 0)),
        ],
        out_specs=pl.BlockSpec((1, 8, D), lambda i: (i, 0, 0)),
        out_shape=jax.ShapeDtypeStruct((GRID, 8, D), jnp.float32),
    )(h, p, inv16, ws, b.reshape(1, D))


def _head_body(ps_ref, w1_ref, b1_ref, w2_ref, b2_ref, out_ref):
    m = jnp.sum(ps_ref[...], axis=(0, 1)).reshape(1, D) * (1.0 / N)
    y = jnp.dot(m, w1_ref[...], preferred_element_type=jnp.float32) + b1_ref[...]
    y = jnp.where(y > 0, y, jnp.exp(y) - 1.0)
    z = jnp.dot(y, w2_ref[...], preferred_element_type=jnp.float32) + b2_ref[...]
    mx = jnp.max(z, axis=0, keepdims=True)
    e = z - mx
    out_ref[...] = e - jnp.log(jnp.sum(jnp.exp(e), axis=0, keepdims=True))


def _head(psums, w1, b1, w2, b2):
    return pl.pallas_call(
        _head_body,
        out_shape=jax.ShapeDtypeStruct((1, N_CLS), jnp.float32),
    )(psums, w1, b1.reshape(1, D), w2, b2.reshape(1, N_CLS))


def kernel(x, edge_index, Ws0, Wn0, b0, Ws1, Wn1, b1, Ws2, Wn2, b2,
           Ws3, Wn3, b3, W_fc1, b_fc1, W_fc2, b_fc2):
    src = edge_index[0].astype(jnp.int32)
    dst = edge_index[1].astype(jnp.int32)
    pad = EPAD - E
    src3 = jnp.concatenate([src, jnp.zeros((pad,), jnp.int32)]).reshape(NW, NCH, C)
    dst3 = jnp.concatenate([dst, jnp.full((pad,), N, jnp.int32)]).reshape(NW, NCH, C)

    pdeg = _deg(dst3)
    g, inv16 = _proj0(x, Wn0, pdeg)

    h = x
    layers = [(Ws0, b0, Wn1), (Ws1, b1, Wn2), (Ws2, b2, Wn3)]
    for (ws, b, wn_next) in layers:
        p = _segsum(g, src3, dst3)
        h, g = _combine(h, p, inv16, ws, b, wn_next)
    p = _segsum(g, src3, dst3)
    psums = _combine_last(h, p, inv16, Ws3, b3)

    return _head(psums, W_fc1, b_fc1, W_fc2, b_fc2)
